# baseline (speedup 1.0000x reference)
import functools
import math

import jax
import jax.numpy as jnp
from jax import lax
from jax.experimental import pallas as pl
from jax.experimental.pallas import tpu as pltpu
from jax.experimental.pallas import tpu_sc as plsc


def _sc_probe(x_hbm, out_hbm, row_v, sem):
    wid = lax.axis_index("s") * 2 + lax.axis_index("c")
    pltpu.async_copy(x_hbm.at[wid], row_v, sem).wait()
    pltpu.sync_copy(row_v, out_hbm.at[wid])


def _run_sc_probe(x2):
    D = x2.shape[1]
    mesh = plsc.VectorSubcoreMesh(core_axis_name="c", subcore_axis_name="s")
    return pl.kernel(
        _sc_probe,
        out_type=jax.ShapeDtypeStruct((32, D), jnp.float32),
        mesh=mesh,
        scratch_types=[
            pltpu.VMEM((D,), jnp.float32),
            pltpu.SemaphoreType.DMA,
        ],
    )(x2[:32])


def _gelu(x):
    theta_x = 1 + jnp.tanh(math.sqrt(2 / math.pi) * (x + 0.044715 * jnp.power(x, 3)))
    return 0.5 * x * theta_x


def _fused_kernel(x_ref, xk_ref, rw_ref, rb_ref, fcb_ref, pjb_ref,
                  fcw_hbm, pjw_hbm, out_ref,
                  acc_ref, w1_ref, w2_ref,
                  s0, s1, s2, s3,
                  *, n_steps, n_total, n_out):
    step = pl.program_id(0)
    part = jnp.sum(x_ref[...], axis=0, keepdims=True)

    @pl.when(step == 0)
    def _init():
        acc_ref[...] = part

    @pl.when(step > 0)
    def _acc():
        acc_ref[...] += part

    @pl.when(step == n_steps - 1)
    def _finish():
        mean_x = acc_ref[...] / n_total
        scores = jnp.dot(mean_x, rw_ref[...],
                         preferred_element_type=jnp.float32) + rb_ref[...]
        m = jnp.max(scores, axis=1, keepdims=True)
        e = jnp.exp(scores - m)
        p = e / jnp.sum(e, axis=1, keepdims=True)
        i0 = jnp.argmax(p, axis=1)[0]
        v0 = jnp.max(p, axis=1)[0]
        col = jax.lax.broadcasted_iota(jnp.int32, p.shape, 1)
        p2 = jnp.where(col == i0, -jnp.inf, p)
        i1 = jnp.argmax(p2, axis=1)[0]
        v1 = jnp.max(p2, axis=1)[0]

        D2 = w1_ref.shape[1] // 2
        F2 = w2_ref.shape[1] // 2
        copies = [
            pltpu.make_async_copy(fcw_hbm.at[i0, pl.ds(0, D2)],
                                  w1_ref.at[0, pl.ds(0, D2)], s0.at[0]),
            pltpu.make_async_copy(fcw_hbm.at[i0, pl.ds(D2, D2)],
                                  w1_ref.at[0, pl.ds(D2, D2)], s0.at[1]),
            pltpu.make_async_copy(fcw_hbm.at[i1, pl.ds(0, D2)],
                                  w1_ref.at[1, pl.ds(0, D2)], s1.at[0]),
            pltpu.make_async_copy(fcw_hbm.at[i1, pl.ds(D2, D2)],
                                  w1_ref.at[1, pl.ds(D2, D2)], s1.at[1]),
            pltpu.make_async_copy(pjw_hbm.at[i0, pl.ds(0, F2)],
                                  w2_ref.at[0, pl.ds(0, F2)], s2.at[0]),
            pltpu.make_async_copy(pjw_hbm.at[i0, pl.ds(F2, F2)],
                                  w2_ref.at[0, pl.ds(F2, F2)], s2.at[1]),
            pltpu.make_async_copy(pjw_hbm.at[i1, pl.ds(0, F2)],
                                  w2_ref.at[1, pl.ds(0, F2)], s3.at[0]),
            pltpu.make_async_copy(pjw_hbm.at[i1, pl.ds(F2, F2)],
                                  w2_ref.at[1, pl.ds(F2, F2)], s3.at[1]),
        ]
        for cp in copies:
            cp.start()
        c0, c1, c2, c3 = copies[0:2], copies[2:4], copies[4:6], copies[6:8]

        xv0 = xk_ref[0]
        xv1 = xk_ref[1]
        b1_0 = fcb_ref[i0]
        b1_1 = fcb_ref[i1]
        b2_0 = pjb_ref[i0]
        b2_1 = pjb_ref[i1]

        for cp in c0:
            cp.wait()
        h0 = _gelu(jnp.dot(xv0, w1_ref[0],
                           preferred_element_type=jnp.float32) + b1_0)
        for cp in c1:
            cp.wait()
        h1 = _gelu(jnp.dot(xv1, w1_ref[1],
                           preferred_element_type=jnp.float32) + b1_1)
        for cp in c2:
            cp.wait()
        y0 = jnp.dot(h0, w2_ref[0], preferred_element_type=jnp.float32) + b2_0
        for cp in c3:
            cp.wait()
        y1 = jnp.dot(h1, w2_ref[1], preferred_element_type=jnp.float32) + b2_1
        row = v0 * y0 + v1 * y1
        out_ref[...] = jnp.broadcast_to(row, (n_out, row.shape[1]))


def kernel(x, fc_w, fc_b, proj_w, proj_b, route_w, route_b):
    B, N, D = x.shape
    E, _, F = fc_w.shape
    K = 2
    x2 = x[0]

    n_steps = 8
    tile = N // n_steps
    out2 = pl.pallas_call(
        functools.partial(_fused_kernel, n_steps=n_steps, n_total=float(N),
                          n_out=N),
        grid=(n_steps,),
        in_specs=[
            pl.BlockSpec((tile, D), lambda s: (s, 0)),
            pl.BlockSpec((K, 1, D), lambda s: (0, 0, 0)),
            pl.BlockSpec((D, E), lambda s: (0, 0)),
            pl.BlockSpec((1, E), lambda s: (0, 0)),
            pl.BlockSpec((E, 1, F), lambda s: (0, 0, 0)),
            pl.BlockSpec((E, 1, D), lambda s: (0, 0, 0)),
            pl.BlockSpec(memory_space=pltpu.HBM),
            pl.BlockSpec(memory_space=pltpu.HBM),
        ],
        out_specs=pl.BlockSpec((N, D), lambda s: (0, 0)),
        out_shape=jax.ShapeDtypeStruct((N, D), jnp.float32),
        scratch_shapes=[
            pltpu.VMEM((1, D), jnp.float32),
            pltpu.VMEM((K, D, F), jnp.float32),
            pltpu.VMEM((K, F, D), jnp.float32),
            pltpu.SemaphoreType.DMA((2,)),
            pltpu.SemaphoreType.DMA((2,)),
            pltpu.SemaphoreType.DMA((2,)),
            pltpu.SemaphoreType.DMA((2,)),
        ],
    )(x2, x2[:K].reshape(K, 1, D), route_w, route_b.reshape(1, E),
      fc_b.reshape(E, 1, F), proj_b.reshape(E, 1, D), fc_w, proj_w)

    probe = _run_sc_probe(x2)
    return (probe[0, 0] * 0.0 + out2)[None]


# speedup vs baseline: 1.9472x; 1.9472x over previous
import functools
import math

import jax
import jax.numpy as jnp
from jax.experimental import pallas as pl
from jax.experimental.pallas import tpu as pltpu


def _gelu(x):
    theta_x = 1 + jnp.tanh(math.sqrt(2 / math.pi) * (x + 0.044715 * jnp.power(x, 3)))
    return 0.5 * x * theta_x


def _fused_kernel(x_ref, xk_ref, rw_ref, rb_ref, fcb_ref, pjb_ref,
                  fcw_hbm, pjw_hbm, out_hbm,
                  acc_ref, w1_ref, w2_ref, bc_ref,
                  s0, s1, s2, s3, so,
                  *, n_steps, n_total, n_out, out_tiles):
    step = pl.program_id(0)
    part = jnp.sum(x_ref[...], axis=0, keepdims=True)

    @pl.when(step == 0)
    def _init():
        acc_ref[...] = part

    @pl.when(step > 0)
    def _acc():
        acc_ref[...] += part

    @pl.when(step == n_steps - 1)
    def _finish():
        mean_x = acc_ref[...] / n_total
        scores = jnp.dot(mean_x, rw_ref[...],
                         preferred_element_type=jnp.float32) + rb_ref[...]
        i0 = jnp.argmax(scores, axis=1)[0]
        col = jax.lax.broadcasted_iota(jnp.int32, scores.shape, 1)
        masked = jnp.where(col == i0, -jnp.inf, scores)
        i1 = jnp.argmax(masked, axis=1)[0]

        F = w2_ref.shape[1]
        F2 = F // 2
        c0 = pltpu.make_async_copy(fcw_hbm.at[i0], w1_ref.at[0], s0)
        c1 = pltpu.make_async_copy(fcw_hbm.at[i1], w1_ref.at[1], s1)
        c2a = pltpu.make_async_copy(pjw_hbm.at[i0, pl.ds(0, F2)],
                                    w2_ref.at[0, pl.ds(0, F2)], s2.at[0])
        c2b = pltpu.make_async_copy(pjw_hbm.at[i0, pl.ds(F2, F2)],
                                    w2_ref.at[0, pl.ds(F2, F2)], s2.at[1])
        c3a = pltpu.make_async_copy(pjw_hbm.at[i1, pl.ds(0, F2)],
                                    w2_ref.at[1, pl.ds(0, F2)], s3.at[0])
        c3b = pltpu.make_async_copy(pjw_hbm.at[i1, pl.ds(F2, F2)],
                                    w2_ref.at[1, pl.ds(F2, F2)], s3.at[1])
        c0.start()
        c1.start()
        c2a.start()
        c2b.start()
        c3a.start()
        c3b.start()

        m = jnp.max(scores, axis=1, keepdims=True)
        e = jnp.exp(scores - m)
        p = e / jnp.sum(e, axis=1, keepdims=True)
        v0 = jnp.max(p, axis=1)[0]
        p2 = jnp.where(col == i0, -jnp.inf, p)
        v1 = jnp.max(p2, axis=1)[0]

        xv0 = xk_ref[0]
        xv1 = xk_ref[1]
        b1_0 = fcb_ref[i0]
        b1_1 = fcb_ref[i1]
        b2_0 = pjb_ref[i0]
        b2_1 = pjb_ref[i1]

        c0.wait()
        h0 = _gelu(jnp.dot(xv0, w1_ref[0],
                           preferred_element_type=jnp.float32) + b1_0)
        c1.wait()
        h1 = _gelu(jnp.dot(xv1, w1_ref[1],
                           preferred_element_type=jnp.float32) + b1_1)
        c2a.wait()
        y0 = jnp.dot(h0[:, :F2], w2_ref[0, :F2],
                     preferred_element_type=jnp.float32) + b2_0
        c3a.wait()
        y1 = jnp.dot(h1[:, :F2], w2_ref[1, :F2],
                     preferred_element_type=jnp.float32) + b2_1
        c2b.wait()
        y0 = y0 + jnp.dot(h0[:, F2:], w2_ref[0, F2:],
                          preferred_element_type=jnp.float32)
        c3b.wait()
        y1 = y1 + jnp.dot(h1[:, F2:], w2_ref[1, F2:],
                          preferred_element_type=jnp.float32)
        row = v0 * y0 + v1 * y1

        rows_per_tile = n_out // out_tiles
        bc_ref[...] = jnp.broadcast_to(row, (rows_per_tile, row.shape[1]))
        outs = [
            pltpu.make_async_copy(
                bc_ref, out_hbm.at[pl.ds(t * rows_per_tile, rows_per_tile)],
                so.at[t])
            for t in range(out_tiles)
        ]
        for cp in outs:
            cp.start()
        for cp in outs:
            cp.wait()


def kernel(x, fc_w, fc_b, proj_w, proj_b, route_w, route_b):
    B, N, D = x.shape
    E, _, F = fc_w.shape
    K = 2
    x2 = x[0]

    n_steps = 8
    tile = N // n_steps
    out_tiles = 8
    out2 = pl.pallas_call(
        functools.partial(_fused_kernel, n_steps=n_steps, n_total=float(N),
                          n_out=N, out_tiles=out_tiles),
        grid=(n_steps,),
        in_specs=[
            pl.BlockSpec((tile, D), lambda s: (s, 0)),
            pl.BlockSpec((K, 1, D), lambda s: (0, 0, 0)),
            pl.BlockSpec((D, E), lambda s: (0, 0)),
            pl.BlockSpec((1, E), lambda s: (0, 0)),
            pl.BlockSpec((E, 1, F), lambda s: (0, 0, 0)),
            pl.BlockSpec((E, 1, D), lambda s: (0, 0, 0)),
            pl.BlockSpec(memory_space=pltpu.HBM),
            pl.BlockSpec(memory_space=pltpu.HBM),
        ],
        out_specs=pl.BlockSpec(memory_space=pltpu.HBM),
        out_shape=jax.ShapeDtypeStruct((N, D), jnp.float32),
        scratch_shapes=[
            pltpu.VMEM((1, D), jnp.float32),
            pltpu.VMEM((K, D, F), jnp.float32),
            pltpu.VMEM((K, F, D), jnp.float32),
            pltpu.VMEM((N // out_tiles, D), jnp.float32),
            pltpu.SemaphoreType.DMA,
            pltpu.SemaphoreType.DMA,
            pltpu.SemaphoreType.DMA((2,)),
            pltpu.SemaphoreType.DMA((2,)),
            pltpu.SemaphoreType.DMA((out_tiles,)),
        ],
    )(x2, x2[:K].reshape(K, 1, D), route_w, route_b.reshape(1, E),
      fc_b.reshape(E, 1, F), proj_b.reshape(E, 1, D), fc_w, proj_w)

    return out2[None]


# speedup vs baseline: 2.0594x; 1.0576x over previous
import functools
import math

import jax
import jax.numpy as jnp
from jax.experimental import pallas as pl
from jax.experimental.pallas import tpu as pltpu


def _gelu(x):
    theta_x = 1 + jnp.tanh(math.sqrt(2 / math.pi) * (x + 0.044715 * jnp.power(x, 3)))
    return 0.5 * x * theta_x


def _fused_kernel(x_hbm, xk_ref, rw_ref, rb_ref, fcb_ref, pjb_ref,
                  fcw_hbm, pjw_hbm, out_hbm,
                  xb_ref, w1_ref, w2_ref, bc_ref,
                  sx, s0, s1, s2, s3, so,
                  *, n_total, n_out, x_chunks, out_tiles):
    rows = x_hbm.shape[0] // x_chunks
    xcopies = [
        pltpu.make_async_copy(x_hbm.at[pl.ds(i * rows, rows)],
                              xb_ref.at[i], sx.at[i])
        for i in range(x_chunks)
    ]
    for cp in xcopies:
        cp.start()
    xcopies[0].wait()
    acc = jnp.sum(xb_ref[0], axis=0, keepdims=True)
    for i in range(1, x_chunks):
        xcopies[i].wait()
        acc = acc + jnp.sum(xb_ref[i], axis=0, keepdims=True)

    mean_x = acc / n_total
    scores = jnp.dot(mean_x, rw_ref[...],
                     preferred_element_type=jnp.float32) + rb_ref[...]
    i0 = jnp.argmax(scores, axis=1)[0]
    col = jax.lax.broadcasted_iota(jnp.int32, scores.shape, 1)
    masked = jnp.where(col == i0, -jnp.inf, scores)
    i1 = jnp.argmax(masked, axis=1)[0]

    F = w2_ref.shape[1]
    F2 = F // 2
    c0 = pltpu.make_async_copy(fcw_hbm.at[i0], w1_ref.at[0], s0)
    c1 = pltpu.make_async_copy(fcw_hbm.at[i1], w1_ref.at[1], s1)
    c2a = pltpu.make_async_copy(pjw_hbm.at[i0, pl.ds(0, F2)],
                                w2_ref.at[0, pl.ds(0, F2)], s2.at[0])
    c2b = pltpu.make_async_copy(pjw_hbm.at[i0, pl.ds(F2, F2)],
                                w2_ref.at[0, pl.ds(F2, F2)], s2.at[1])
    c3a = pltpu.make_async_copy(pjw_hbm.at[i1, pl.ds(0, F2)],
                                w2_ref.at[1, pl.ds(0, F2)], s3.at[0])
    c3b = pltpu.make_async_copy(pjw_hbm.at[i1, pl.ds(F2, F2)],
                                w2_ref.at[1, pl.ds(F2, F2)], s3.at[1])
    c0.start()
    c1.start()
    c2a.start()
    c2b.start()
    c3a.start()
    c3b.start()

    m = jnp.max(scores, axis=1, keepdims=True)
    e = jnp.exp(scores - m)
    p = e / jnp.sum(e, axis=1, keepdims=True)
    v0 = jnp.max(p, axis=1)[0]
    p2 = jnp.where(col == i0, -jnp.inf, p)
    v1 = jnp.max(p2, axis=1)[0]

    xv0 = xk_ref[0]
    xv1 = xk_ref[1]
    b1_0 = fcb_ref[i0]
    b1_1 = fcb_ref[i1]
    b2_0 = pjb_ref[i0]
    b2_1 = pjb_ref[i1]

    c0.wait()
    h0 = _gelu(jnp.dot(xv0, w1_ref[0],
                       preferred_element_type=jnp.float32) + b1_0)
    c1.wait()
    h1 = _gelu(jnp.dot(xv1, w1_ref[1],
                       preferred_element_type=jnp.float32) + b1_1)
    c2a.wait()
    y0 = jnp.dot(h0[:, :F2], w2_ref[0, :F2],
                 preferred_element_type=jnp.float32) + b2_0
    c3a.wait()
    y1 = jnp.dot(h1[:, :F2], w2_ref[1, :F2],
                 preferred_element_type=jnp.float32) + b2_1
    c2b.wait()
    y0 = y0 + jnp.dot(h0[:, F2:], w2_ref[0, F2:],
                      preferred_element_type=jnp.float32)
    c3b.wait()
    y1 = y1 + jnp.dot(h1[:, F2:], w2_ref[1, F2:],
                      preferred_element_type=jnp.float32)
    row = v0 * y0 + v1 * y1

    rows_per_tile = n_out // out_tiles
    bc_ref[...] = jnp.broadcast_to(row, (rows_per_tile, row.shape[1]))
    outs = [
        pltpu.make_async_copy(
            bc_ref, out_hbm.at[pl.ds(t * rows_per_tile, rows_per_tile)],
            so.at[t])
        for t in range(out_tiles)
    ]
    for cp in outs:
        cp.start()
    for cp in outs:
        cp.wait()


def kernel(x, fc_w, fc_b, proj_w, proj_b, route_w, route_b):
    B, N, D = x.shape
    E, _, F = fc_w.shape
    K = 2
    x2 = x[0]

    x_chunks = 8
    out_tiles = 8
    out2 = pl.pallas_call(
        functools.partial(_fused_kernel, n_total=float(N), n_out=N,
                          x_chunks=x_chunks, out_tiles=out_tiles),
        grid=(1,),
        in_specs=[
            pl.BlockSpec(memory_space=pltpu.HBM),
            pl.BlockSpec((K, 1, D), lambda s: (0, 0, 0)),
            pl.BlockSpec((D, E), lambda s: (0, 0)),
            pl.BlockSpec((1, E), lambda s: (0, 0)),
            pl.BlockSpec((E, 1, F), lambda s: (0, 0, 0)),
            pl.BlockSpec((E, 1, D), lambda s: (0, 0, 0)),
            pl.BlockSpec(memory_space=pltpu.HBM),
            pl.BlockSpec(memory_space=pltpu.HBM),
        ],
        out_specs=pl.BlockSpec(memory_space=pltpu.HBM),
        out_shape=jax.ShapeDtypeStruct((N, D), jnp.float32),
        scratch_shapes=[
            pltpu.VMEM((x_chunks, N // x_chunks, D), jnp.float32),
            pltpu.VMEM((K, D, F), jnp.float32),
            pltpu.VMEM((K, F, D), jnp.float32),
            pltpu.VMEM((N // out_tiles, D), jnp.float32),
            pltpu.SemaphoreType.DMA((x_chunks,)),
            pltpu.SemaphoreType.DMA,
            pltpu.SemaphoreType.DMA,
            pltpu.SemaphoreType.DMA((2,)),
            pltpu.SemaphoreType.DMA((2,)),
            pltpu.SemaphoreType.DMA((out_tiles,)),
        ],
    )(x2, x2[:K].reshape(K, 1, D), route_w, route_b.reshape(1, E),
      fc_b.reshape(E, 1, F), proj_b.reshape(E, 1, D), fc_w, proj_w)

    return out2[None]


# speedup vs baseline: 2.4945x; 1.2113x over previous
import functools
import math

import jax
import jax.numpy as jnp
from jax.experimental import pallas as pl
from jax.experimental.pallas import tpu as pltpu


def _gelu(x):
    theta_x = 1 + jnp.tanh(math.sqrt(2 / math.pi) * (x + 0.044715 * jnp.power(x, 3)))
    return 0.5 * x * theta_x


def _fused_kernel(x_hbm, rw_ref, rb_ref, fcb_ref, pjb_ref,
                  fcw_hbm, pjw_hbm, out_hbm,
                  xb_ref, w1_ref, w2_ref, bc_ref,
                  sx, s0, s1, s2, s3, so,
                  *, n_total, n_out, x_chunks, out_tiles):
    rows = x_hbm.shape[0] // x_chunks
    xcopies = [
        pltpu.make_async_copy(x_hbm.at[pl.ds(i * rows, rows)],
                              xb_ref.at[i], sx.at[i])
        for i in range(x_chunks)
    ]
    for cp in xcopies:
        cp.start()
    xcopies[0].wait()
    acc = jnp.sum(xb_ref[0], axis=0, keepdims=True)
    for i in range(1, x_chunks):
        xcopies[i].wait()
        acc = acc + jnp.sum(xb_ref[i], axis=0, keepdims=True)

    mean_x = acc / n_total
    scores = jnp.dot(mean_x, rw_ref[...],
                     preferred_element_type=jnp.float32) + rb_ref[...]
    i0 = jnp.argmax(scores, axis=1)[0]
    col = jax.lax.broadcasted_iota(jnp.int32, scores.shape, 1)
    masked = jnp.where(col == i0, -jnp.inf, scores)
    i1 = jnp.argmax(masked, axis=1)[0]

    F = w2_ref.shape[1]
    F2 = F // 2
    c0 = pltpu.make_async_copy(fcw_hbm.at[i0], w1_ref.at[0], s0)
    c1 = pltpu.make_async_copy(fcw_hbm.at[i1], w1_ref.at[1], s1)
    c2a = pltpu.make_async_copy(pjw_hbm.at[i0, pl.ds(0, F2)],
                                w2_ref.at[0, pl.ds(0, F2)], s2.at[0])
    c2b = pltpu.make_async_copy(pjw_hbm.at[i0, pl.ds(F2, F2)],
                                w2_ref.at[0, pl.ds(F2, F2)], s2.at[1])
    c3a = pltpu.make_async_copy(pjw_hbm.at[i1, pl.ds(0, F2)],
                                w2_ref.at[1, pl.ds(0, F2)], s3.at[0])
    c3b = pltpu.make_async_copy(pjw_hbm.at[i1, pl.ds(F2, F2)],
                                w2_ref.at[1, pl.ds(F2, F2)], s3.at[1])
    c0.start()
    c1.start()
    c2a.start()
    c2b.start()
    c3a.start()
    c3b.start()

    m = jnp.max(scores, axis=1, keepdims=True)
    e = jnp.exp(scores - m)
    p = e / jnp.sum(e, axis=1, keepdims=True)
    v0 = jnp.max(p, axis=1)[0]
    p2 = jnp.where(col == i0, -jnp.inf, p)
    v1 = jnp.max(p2, axis=1)[0]

    xv0 = xb_ref[0, 0:1, :]
    xv1 = xb_ref[0, 1:2, :]
    b1_0 = fcb_ref[pl.ds(i0, 1), :]
    b1_1 = fcb_ref[pl.ds(i1, 1), :]
    b2_0 = pjb_ref[pl.ds(i0, 1), :]
    b2_1 = pjb_ref[pl.ds(i1, 1), :]

    c0.wait()
    h0 = _gelu(jnp.dot(xv0, w1_ref[0],
                       preferred_element_type=jnp.float32) + b1_0)
    c1.wait()
    h1 = _gelu(jnp.dot(xv1, w1_ref[1],
                       preferred_element_type=jnp.float32) + b1_1)
    c2a.wait()
    y0 = jnp.dot(h0[:, :F2], w2_ref[0, :F2],
                 preferred_element_type=jnp.float32) + b2_0
    c3a.wait()
    y1 = jnp.dot(h1[:, :F2], w2_ref[1, :F2],
                 preferred_element_type=jnp.float32) + b2_1
    c2b.wait()
    y0 = y0 + jnp.dot(h0[:, F2:], w2_ref[0, F2:],
                      preferred_element_type=jnp.float32)
    c3b.wait()
    y1 = y1 + jnp.dot(h1[:, F2:], w2_ref[1, F2:],
                      preferred_element_type=jnp.float32)
    row = v0 * y0 + v1 * y1

    rows_per_tile = n_out // out_tiles
    bc_ref[...] = jnp.broadcast_to(row, (rows_per_tile, row.shape[1]))
    outs = [
        pltpu.make_async_copy(
            bc_ref, out_hbm.at[pl.ds(t * rows_per_tile, rows_per_tile)],
            so.at[t])
        for t in range(out_tiles)
    ]
    for cp in outs:
        cp.start()
    for cp in outs:
        cp.wait()


def kernel(x, fc_w, fc_b, proj_w, proj_b, route_w, route_b):
    B, N, D = x.shape
    E, _, F = fc_w.shape
    K = 2
    x2 = x[0]

    x_chunks = 8
    out_tiles = 8
    out2 = pl.pallas_call(
        functools.partial(_fused_kernel, n_total=float(N), n_out=N,
                          x_chunks=x_chunks, out_tiles=out_tiles),
        grid=(1,),
        in_specs=[
            pl.BlockSpec(memory_space=pltpu.HBM),
            pl.BlockSpec((D, E), lambda s: (0, 0)),
            pl.BlockSpec((1, E), lambda s: (0, 0)),
            pl.BlockSpec((E, F), lambda s: (0, 0)),
            pl.BlockSpec((E, D), lambda s: (0, 0)),
            pl.BlockSpec(memory_space=pltpu.HBM),
            pl.BlockSpec(memory_space=pltpu.HBM),
        ],
        out_specs=pl.BlockSpec(memory_space=pltpu.HBM),
        out_shape=jax.ShapeDtypeStruct((N, D), jnp.float32),
        scratch_shapes=[
            pltpu.VMEM((x_chunks, N // x_chunks, D), jnp.float32),
            pltpu.VMEM((K, D, F), jnp.float32),
            pltpu.VMEM((K, F, D), jnp.float32),
            pltpu.VMEM((N // out_tiles, D), jnp.float32),
            pltpu.SemaphoreType.DMA((x_chunks,)),
            pltpu.SemaphoreType.DMA,
            pltpu.SemaphoreType.DMA,
            pltpu.SemaphoreType.DMA((2,)),
            pltpu.SemaphoreType.DMA((2,)),
            pltpu.SemaphoreType.DMA((out_tiles,)),
        ],
    )(x2, route_w, route_b.reshape(1, E), fc_b, proj_b, fc_w, proj_w)

    return out2[None]
